# Initial kernel scaffold; baseline (speedup 1.0000x reference)
import jax
import jax.numpy as jnp
from jax.experimental import pallas as pl


def kernel(x, edge_index, etypes, W_rel, loop_weight, h_bias, ln_scale, ln_bias):
    raise NotImplementedError("write your pallas kernel here")


# speedup vs baseline: 17.3992x; 17.3992x over previous
import functools

import jax
import jax.numpy as jnp
from jax import lax
from jax.experimental import pallas as pl
from jax.experimental.pallas import tpu as pltpu
from jax.experimental.pallas import tpu_sc as plsc

N = 10000
E = 320000
D = 128
R = 20
RP = R + 2
ZROW = (R + 1) * N

NC = 2
NS = 16
NW = NC * NS
EW = E // NW
CH = 128
EPAD = 10112
NCH = EPAD // CH
NPAD = 10112
RS = NPAD // NS

BN = 400
NB = N // BN


def _mm_body(x_ref, w_ref, y_ref):
    for r in range(RP):
        y_ref[r] = jnp.dot(x_ref[...], w_ref[r],
                           preferred_element_type=jnp.float32)


def _typed_matmuls(x, wall):
    return pl.pallas_call(
        _mm_body,
        grid=(NB,),
        in_specs=[
            pl.BlockSpec((BN, D), lambda i: (i, 0)),
            pl.BlockSpec((RP, D, D), lambda i: (0, 0, 0)),
        ],
        out_specs=pl.BlockSpec((RP, BN, D), lambda i: (0, i, 0)),
        out_shape=jax.ShapeDtypeStruct((RP, N, D), jnp.float32),
    )(x, wall)


def _sc_body(y_hbm, src_hbm, dst_hbm, et_hbm, out_hbm,
             src_v, dst_v, et_v, idx_buf, dst_buf, rows_v, acc, sem):
    cid = lax.axis_index("c")
    sid = lax.axis_index("s")
    wid = cid * NS + sid
    base_e = wid * EW

    pltpu.sync_copy(src_hbm.at[pl.ds(base_e, EW)], src_v.at[pl.ds(0, EW)])
    pltpu.sync_copy(dst_hbm.at[pl.ds(base_e, EW)], dst_v.at[pl.ds(0, EW)])
    pltpu.sync_copy(et_hbm.at[pl.ds(base_e, EW)], et_v.at[pl.ds(0, EW)])
    for t in range((EPAD - EW) // 16):
        o = EW + t * 16
        src_v[pl.ds(o, 16)] = jnp.full((16,), ZROW, dtype=jnp.int32)
        et_v[pl.ds(o, 16)] = jnp.zeros((16,), dtype=jnp.int32)
        dst_v[pl.ds(o, 16)] = jnp.zeros((16,), dtype=jnp.int32)

    def _zrow(i, c):
        for j in range(D // 16):
            rows_v[i, pl.ds(j * 16, 16)] = jnp.zeros((16,), jnp.float32)
        return c
    lax.fori_loop(0, CH, _zrow, 0)
    rbase = sid * RS
    for t in range(RS // CH):
        pltpu.sync_copy(rows_v, acc.at[pl.ds(rbase + t * CH, CH)])
    rtail = RS % CH
    if rtail:
        pltpu.sync_copy(rows_v.at[pl.ds(0, rtail)],
                        acc.at[pl.ds(rbase + (RS // CH) * CH, rtail)])
    plsc.subcore_barrier()

    def _chunk(c, carry):
        off = c * CH
        for j in range(CH // 16):
            o = off + j * 16
            e16 = et_v[pl.ds(o, 16)]
            s16 = src_v[pl.ds(o, 16)]
            idx_buf[pl.ds(j * 16, 16)] = e16 * N + s16
            dst_buf[pl.ds(j * 16, 16)] = dst_v[pl.ds(o, 16)]
        pltpu.async_copy(y_hbm.at[idx_buf], rows_v, sem).wait()
        pltpu.sync_copy(rows_v, acc.at[dst_buf], add=True)
        return carry
    lax.fori_loop(0, NCH, _chunk, 0)
    plsc.subcore_barrier()

    pltpu.sync_copy(acc.at[pl.ds(rbase, RS)],
                    out_hbm.at[pl.ds(cid * NPAD + rbase, RS)])


def _sc_aggregate(yflat, src, dst, etypes):
    mesh = plsc.VectorSubcoreMesh(core_axis_name="c", subcore_axis_name="s")
    fn = functools.partial(
        pl.kernel,
        mesh=mesh,
        out_type=jax.ShapeDtypeStruct((NC * NPAD, D), jnp.float32),
        scratch_types=[
            pltpu.VMEM((EPAD,), jnp.int32),
            pltpu.VMEM((EPAD,), jnp.int32),
            pltpu.VMEM((EPAD,), jnp.int32),
            pltpu.VMEM((CH,), jnp.int32),
            pltpu.VMEM((CH,), jnp.int32),
            pltpu.VMEM((CH, D), jnp.float32),
            pltpu.VMEM_SHARED((NPAD, D), jnp.float32),
            pltpu.SemaphoreType.DMA,
        ],
    )(_sc_body)
    return fn(yflat, src, dst, etypes)


def _ln_body(p_ref, yl_ref, hb_ref, ls_ref, lb_ref, o_ref):
    h = p_ref[0] + p_ref[1]
    mean = jnp.mean(h, axis=-1, keepdims=True)
    cent = h - mean
    var = jnp.mean(cent * cent, axis=-1, keepdims=True)
    hn = cent * lax.rsqrt(var + 1e-5)
    o_ref[...] = (hn * ls_ref[0] + lb_ref[0] + hb_ref[0]) + yl_ref[...]


def _ln_combine(parts, yloop, h_bias, ln_scale, ln_bias):
    return pl.pallas_call(
        _ln_body,
        grid=(NB,),
        in_specs=[
            pl.BlockSpec((NC, BN, D), lambda i: (0, i, 0)),
            pl.BlockSpec((BN, D), lambda i: (i, 0)),
            pl.BlockSpec((1, D), lambda i: (0, 0)),
            pl.BlockSpec((1, D), lambda i: (0, 0)),
            pl.BlockSpec((1, D), lambda i: (0, 0)),
        ],
        out_specs=pl.BlockSpec((BN, D), lambda i: (i, 0)),
        out_shape=jax.ShapeDtypeStruct((N, D), jnp.float32),
    )(parts, yloop, h_bias, ln_scale, ln_bias)


@jax.jit
def kernel(x, edge_index, etypes, W_rel, loop_weight, h_bias, ln_scale,
           ln_bias):
    wall = jnp.concatenate(
        [W_rel, loop_weight[None], jnp.zeros((1, D, D), jnp.float32)], axis=0)
    y = _typed_matmuls(x, wall)
    yflat = y.reshape(RP * N, D)
    parts = _sc_aggregate(yflat, edge_index[0], edge_index[1], etypes)
    out = _ln_combine(parts.reshape(NC, NPAD, D), y[R],
                      h_bias.reshape(1, D), ln_scale.reshape(1, D),
                      ln_bias.reshape(1, D))
    return out
